# baseline (speedup 1.0000x reference)
import functools

import jax
import jax.numpy as jnp
from jax import lax
from jax.experimental import pallas as pl
from jax.experimental.pallas import tpu as pltpu
from jax.experimental.pallas import tpu_sc as plsc

NC, NS, L = 2, 16, 16
NW = NC * NS

_N = 10000
_NPAD = 10240
_E = 320000
_EPAD = 327680
_CH = 64
_D = 128
_H = 256
_M = 65536
_MW = _M // NW
_B = 512
_NB = _NPAD // _B

def _mesh():
    return plsc.VectorSubcoreMesh(core_axis_name="c", subcore_axis_name="s",
                                  num_cores=NC, num_subcores=NS)
_sc_params = pltpu.CompilerParams(needs_layout_passes=False)


def _sc_degree(dst_pad):
    ew = _EPAD // NW

    @functools.partial(
        pl.kernel,
        out_type=jax.ShapeDtypeStruct((NW, _NPAD), jnp.float32),
        mesh=_mesh(),
        compiler_params=_sc_params,
        scratch_types=[
            pltpu.VMEM((_NPAD,), jnp.float32),
            pltpu.VMEM((ew,), jnp.int32),
        ],
    )
    def deg_kernel(dst_hbm, out_hbm, deg_v, idx_v):
        c = lax.axis_index("c")
        s = lax.axis_index("s")
        w = c * NS + s
        zeros16 = jnp.zeros((L,), jnp.float32)

        def zero_body(i, carry):
            deg_v[pl.ds(i * L, L)] = zeros16
            return carry

        lax.fori_loop(0, _NPAD // L, zero_body, 0)
        pltpu.sync_copy(dst_hbm.at[pl.ds(w * ew, ew)], idx_v)
        ones16 = jnp.ones((L,), jnp.float32)

        def add_body(j, carry):
            idx = idx_v[pl.ds(j * L, L)]
            plsc.addupdate_scatter(deg_v, [idx], ones16)
            return carry

        lax.fori_loop(0, ew // L, add_body, 0)
        pltpu.sync_copy(deg_v, out_hbm.at[w])

    return deg_kernel(dst_pad)


def _pack_bf16(lo, hi):
    lo16 = jax.lax.bitcast_convert_type(lo.astype(jnp.bfloat16), jnp.uint16)
    hi16 = jax.lax.bitcast_convert_type(hi.astype(jnp.bfloat16), jnp.uint16)
    return lo16.astype(jnp.int32) | (hi16.astype(jnp.int32) << 16)


def _unpack_bf16(w):
    lo = jax.lax.bitcast_convert_type(w << 16, jnp.float32)
    hi = jax.lax.bitcast_convert_type(w & jnp.int32(-65536), jnp.float32)
    return lo, hi


def _sc_scatter(table, src_ck, dst_ck, *, chunks_per_core, core_chunk_stride,
                src_3d, F=_D):
    cpt = chunks_per_core // NS
    rpt = _NPAD // NS
    G = 16
    n_super = cpt // G
    PW = F // 2
    params = pltpu.CompilerParams(
        needs_layout_passes=False, use_tc_tiling_on_sc=False)

    @functools.partial(
        pl.kernel,
        out_type=jax.ShapeDtypeStruct((NC, _NPAD, F), jnp.float32),
        mesh=_mesh(),
        compiler_params=params,
        scratch_types=[
            pltpu.VMEM_SHARED((_NPAD, F), jnp.float32),
            pltpu.VMEM((_CH, PW), jnp.int32),
            pltpu.VMEM((_CH, PW), jnp.int32),
            pltpu.VMEM((_CH, PW), jnp.int32),
            pltpu.VMEM((_CH, PW), jnp.int32),
            pltpu.VMEM((_CH, F), jnp.float32),
            pltpu.VMEM((_CH, F), jnp.float32),
            pltpu.VMEM((G, _CH), jnp.int32),
            pltpu.VMEM((G, _CH), jnp.int32),
            pltpu.SemaphoreType.DMA,
            pltpu.SemaphoreType.DMA,
            pltpu.SemaphoreType.DMA,
            pltpu.SemaphoreType.DMA,
            pltpu.SemaphoreType.DMA,
            pltpu.SemaphoreType.DMA,
        ],
    )
    def scat_kernel(table_hbm, src_hbm, dst_hbm, out_hbm, acc, ib0, ib1, ib2,
                    ib3, fb0, fb1, si, di, g0, g1, g2, g3, s0, s1):
        c = lax.axis_index("c")
        s = lax.axis_index("s")
        ibufs = [ib0, ib1, ib2, ib3]
        fbufs = [fb0, fb1]
        gsem = [g0, g1, g2, g3]
        ssem = [s0, s1]
        zeros16 = jnp.zeros((L,), jnp.float32)

        def zrow(k, carry):
            i = k // (F // L)
            j = k % (F // L)
            fb0[i, pl.ds(j * L, L)] = zeros16
            return carry

        lax.fori_loop(0, _CH * (F // L), zrow, 0)

        def zacc(i, carry):
            pltpu.sync_copy(fb0, acc.at[pl.ds(s * rpt + i * _CH, _CH)])
            return carry

        lax.fori_loop(0, rpt // _CH, zacc, 0)
        plsc.subcore_barrier()

        ck0 = c * core_chunk_stride + s * cpt
        ngrp = PW // L

        def gath(p):
            pltpu.async_copy(table_hbm.at[si.at[p]], ibufs[p % 4],
                             gsem[p % 4])

        def gath_wait(p):
            pltpu.make_async_copy(table_hbm.at[si.at[p]], ibufs[p % 4],
                                  gsem[p % 4]).wait()

        def scat_add(p):
            pltpu.async_copy(fbufs[p % 2], acc.at[di.at[p]], ssem[p % 2],
                             add=True)

        def scat_wait(p):
            pltpu.make_async_copy(fbufs[p % 2], acc.at[di.at[p]],
                                  ssem[p % 2]).wait()

        def convert(p):
            ib = ibufs[p % 4]
            fb = fbufs[p % 2]

            def cbody(r, carry):
                for k in range(ngrp):
                    w = ib[r, pl.ds(k * L, L)]
                    lo, hi = _unpack_bf16(w)
                    fb[r, pl.ds(k * L, L)] = lo
                    fb[r, pl.ds(PW + k * L, L)] = hi
                return carry

            lax.fori_loop(0, _CH, cbody, 0, unroll=2)

        def super_body(u, carry):
            base = ck0 + u * G
            if src_3d:
                pltpu.sync_copy(src_hbm.at[c, pl.ds(base, G)], si)
            else:
                pltpu.sync_copy(src_hbm.at[pl.ds(base, G)], si)
            pltpu.sync_copy(dst_hbm.at[pl.ds(base, G)], di)
            for q in range(3):
                gath(q)
            for p in range(G):
                gath_wait(p)
                if p >= 2:
                    scat_wait(p - 2)
                convert(p)
                scat_add(p)
                if p + 3 < G:
                    gath(p + 3)
            scat_wait(G - 2)
            scat_wait(G - 1)
            return carry

        lax.fori_loop(0, n_super, super_body, 0)
        plsc.subcore_barrier()
        pltpu.sync_copy(acc.at[pl.ds(s * rpt, rpt)],
                        out_hbm.at[c, pl.ds(s * rpt, rpt)])

    return scat_kernel(table, src_ck, dst_ck)


def _tc_stage1(deg_parts, xpad, W1):
    def body(deg_ref, x_ref, w_ref, y_ref, dinv_ref):
        deg = jnp.sum(deg_ref[...], axis=0) + 1.0
        dinv = lax.rsqrt(deg)
        xw = jnp.dot(x_ref[...], w_ref[...],
                     preferred_element_type=jnp.float32)
        y = dinv[:, None] * xw
        y_ref[...] = _pack_bf16(y[:, :_D // 2], y[:, _D // 2:])[None]
        dinv_ref[...] = dinv[None, :]

    return pl.pallas_call(
        body,
        grid=(2, _NB),
        in_specs=[
            pl.BlockSpec((NW, _B), lambda h, i: (0, i)),
            pl.BlockSpec((_B, _D), lambda h, i: (i, 0)),
            pl.BlockSpec((_D, _D), lambda h, i: (0, h)),
        ],
        out_specs=[
            pl.BlockSpec((1, _B, _D // 2), lambda h, i: (h, i, 0)),
            pl.BlockSpec((1, _B), lambda h, i: (0, i)),
        ],
        out_shape=[
            jax.ShapeDtypeStruct((NC, _NPAD, _D // 2), jnp.int32),
            jax.ShapeDtypeStruct((1, _NPAD), jnp.float32),
        ],
    )(deg_parts, xpad, W1)


def _tc_stage2(agg1, y1, dinv, b1r, W2):
    F2 = _D // 2

    def body(a_ref, y_ref, dinv_ref, b_ref, w_ref, o_ref):
        dv = dinv_ref[0]
        b = b_ref[0]
        y0lo, y0hi = _unpack_bf16(y_ref[0])
        y1lo, y1hi = _unpack_bf16(y_ref[1])
        y0 = jnp.concatenate([y0lo, y0hi], axis=-1)
        y1 = jnp.concatenate([y1lo, y1hi], axis=-1)
        h_lo = jnp.maximum(
            dv[:, None] * (a_ref[0] + y0) + b[None, :_D], 0.0)
        h_hi = jnp.maximum(
            dv[:, None] * (a_ref[1] + y1) + b[None, _D:], 0.0)
        xw = (jnp.dot(h_lo, w_ref[0, :_D], preferred_element_type=jnp.float32)
              + jnp.dot(h_hi, w_ref[0, _D:],
                        preferred_element_type=jnp.float32))
        y2 = dv[:, None] * xw
        o_ref[...] = _pack_bf16(y2[:, :F2 // 2], y2[:, F2 // 2:])[None]

    return pl.pallas_call(
        body,
        grid=(2, _NB),
        in_specs=[
            pl.BlockSpec((NC, _B, _D), lambda h, i: (0, i, 0)),
            pl.BlockSpec((NC, _B, _D // 2), lambda h, i: (0, i, 0)),
            pl.BlockSpec((1, _B), lambda h, i: (0, i)),
            pl.BlockSpec((1, _H), lambda h, i: (0, 0)),
            pl.BlockSpec((1, _H, F2), lambda h, i: (h, 0, 0)),
        ],
        out_specs=pl.BlockSpec((1, _B, F2 // 2), lambda h, i: (h, i, 0)),
        out_shape=jax.ShapeDtypeStruct((NC, _NPAD, F2 // 2), jnp.int32),
    )(agg1, y1, dinv, b1r, W2)


def _tc_stage3(agg2, y2, dinv, b2r, wlr, blv):
    def body(a_ref, y_ref, dinv_ref, b_ref, wl_ref, bl_ref, s_ref):
        dv = dinv_ref[0]
        y0lo, y0hi = _unpack_bf16(y_ref[0])
        y1lo, y1hi = _unpack_bf16(y_ref[1])
        agg = jnp.concatenate(
            [a_ref[0] + jnp.concatenate([y0lo, y0hi], axis=-1),
             a_ref[1] + jnp.concatenate([y1lo, y1hi], axis=-1)], axis=-1)
        h2 = dv[:, None] * agg + b_ref[0][None, :]
        s = lax.dot_general(wl_ref[...], h2, (((1,), (1,)), ((), ())),
                            preferred_element_type=jnp.float32)
        s_ref[...] = s + bl_ref[...]

    return pl.pallas_call(
        body,
        grid=(_NB,),
        in_specs=[
            pl.BlockSpec((NC, _B, _D // 2), lambda i: (0, i, 0)),
            pl.BlockSpec((NC, _B, _D // 4), lambda i: (0, i, 0)),
            pl.BlockSpec((1, _B), lambda i: (0, i)),
            pl.BlockSpec((1, _D), lambda i: (0, 0)),
            pl.BlockSpec((2, _D), lambda i: (0, 0)),
            pl.BlockSpec((2, 1), lambda i: (0, 0)),
        ],
        out_specs=pl.BlockSpec((2, _B), lambda i: (0, i)),
        out_shape=jax.ShapeDtypeStruct((2, _NPAD), jnp.float32),
    )(agg2, y2, dinv, b2r, wlr, blv)


def _sc_mask(s_pair, m0, m1):
    @functools.partial(
        pl.kernel,
        out_type=jax.ShapeDtypeStruct((_M,), jnp.float32),
        mesh=_mesh(),
        compiler_params=_sc_params,
        scratch_types=[
            pltpu.VMEM((2, _NPAD), jnp.float32),
            pltpu.VMEM((_MW,), jnp.int32),
            pltpu.VMEM((_MW,), jnp.int32),
            pltpu.VMEM((_MW,), jnp.float32),
        ],
    )
    def mask_kernel(s_hbm, m0_hbm, m1_hbm, out_hbm, s_v, i0_v, i1_v, o_v):
        c = lax.axis_index("c")
        s = lax.axis_index("s")
        w = c * NS + s
        pltpu.sync_copy(s_hbm, s_v)
        pltpu.sync_copy(m0_hbm.at[pl.ds(w * _MW, _MW)], i0_v)
        pltpu.sync_copy(m1_hbm.at[pl.ds(w * _MW, _MW)], i1_v)
        z16 = jnp.zeros((L,), jnp.int32)
        o16 = jnp.ones((L,), jnp.int32)

        def body(j, carry):
            i0 = i0_v[pl.ds(j * L, L)]
            i1 = i1_v[pl.ds(j * L, L)]
            a = plsc.load_gather(s_v, [z16, i0])
            b = plsc.load_gather(s_v, [o16, i1])
            o_v[pl.ds(j * L, L)] = 1.0 / (1.0 + jnp.exp(-(a + b)))
            return carry

        lax.fori_loop(0, _MW // L, body, 0)
        pltpu.sync_copy(o_v, out_hbm.at[pl.ds(w * _MW, _MW)])

    return mask_kernel(s_pair, m0, m1)


def kernel(g, features, mask, W1, b1, W2, b2, Wl, bl):
    src = g[0].astype(jnp.int32)
    dst = g[1].astype(jnp.int32)
    padidx = jnp.full((_EPAD - _E,), _N, jnp.int32)
    src_p = jnp.concatenate([src, padidx])
    dst_p = jnp.concatenate([dst, padidx])
    src2 = jnp.stack([src_p, src_p + _NPAD])
    xpad = jnp.pad(features, ((0, _NPAD - _N), (0, 0)))

    n_ck = _EPAD // _CH
    _hbm = lambda a: pltpu.with_memory_space_constraint(a, pltpu.HBM)
    dst_ck = _hbm(dst_p.reshape(n_ck, _CH))
    src2_ck = _hbm(src2.reshape(NC, n_ck, _CH))
    src_ck = _hbm(src_p.reshape(n_ck, _CH))

    deg_parts = _sc_degree(dst_p)
    y1, dinv = _tc_stage1(deg_parts, xpad, W1)
    agg1 = _sc_scatter(y1.reshape(NC * _NPAD, _D // 2), src2_ck, dst_ck,
                       chunks_per_core=n_ck, core_chunk_stride=0,
                       src_3d=True)
    W2r = W2.reshape(_H, 2, _D // 2).transpose(1, 0, 2)
    y2 = _tc_stage2(agg1, y1, dinv, b1.reshape(1, _H), W2r)
    agg2 = _sc_scatter(y2.reshape(NC * _NPAD, _D // 4), src2_ck, dst_ck,
                       chunks_per_core=n_ck, core_chunk_stride=0,
                       src_3d=True, F=_D // 2)
    wlr = Wl[:, 0].reshape(2, _D)
    blv = jnp.pad(bl, (0, 1)).reshape(2, 1)
    s_pair = _tc_stage3(agg2, y2, dinv, b2.reshape(1, _D), wlr, blv)
    out = _sc_mask(s_pair, mask[:, 0].astype(jnp.int32),
                   mask[:, 1].astype(jnp.int32))
    return out.reshape(_M, 1)


# speedup vs baseline: 1.0151x; 1.0151x over previous
import functools

import jax
import jax.numpy as jnp
from jax import lax
from jax.experimental import pallas as pl
from jax.experimental.pallas import tpu as pltpu
from jax.experimental.pallas import tpu_sc as plsc

NC, NS, L = 2, 16, 16
NW = NC * NS

_N = 10000
_NPAD = 10240
_E = 320000
_EPAD = 327680
_CH = 80
_D = 128
_H = 256
_M = 65536
_MW = _M // NW
_B = 512
_NB = _NPAD // _B

def _mesh():
    return plsc.VectorSubcoreMesh(core_axis_name="c", subcore_axis_name="s",
                                  num_cores=NC, num_subcores=NS)
_sc_params = pltpu.CompilerParams(needs_layout_passes=False)


def _sc_degree(dst_pad):
    ew = _EPAD // NW

    @functools.partial(
        pl.kernel,
        out_type=jax.ShapeDtypeStruct((NW, _NPAD), jnp.float32),
        mesh=_mesh(),
        compiler_params=_sc_params,
        scratch_types=[
            pltpu.VMEM((_NPAD,), jnp.float32),
            pltpu.VMEM((ew,), jnp.int32),
        ],
    )
    def deg_kernel(dst_hbm, out_hbm, deg_v, idx_v):
        c = lax.axis_index("c")
        s = lax.axis_index("s")
        w = c * NS + s
        zeros16 = jnp.zeros((L,), jnp.float32)

        def zero_body(i, carry):
            deg_v[pl.ds(i * L, L)] = zeros16
            return carry

        lax.fori_loop(0, _NPAD // L, zero_body, 0)
        pltpu.sync_copy(dst_hbm.at[pl.ds(w * ew, ew)], idx_v)
        ones16 = jnp.ones((L,), jnp.float32)

        def add_body(j, carry):
            idx = idx_v[pl.ds(j * L, L)]
            plsc.addupdate_scatter(deg_v, [idx], ones16)
            return carry

        lax.fori_loop(0, ew // L, add_body, 0)
        pltpu.sync_copy(deg_v, out_hbm.at[w])

    return deg_kernel(dst_pad)


def _pack_bf16(lo, hi):
    lo16 = jax.lax.bitcast_convert_type(lo.astype(jnp.bfloat16), jnp.uint16)
    hi16 = jax.lax.bitcast_convert_type(hi.astype(jnp.bfloat16), jnp.uint16)
    return lo16.astype(jnp.int32) | (hi16.astype(jnp.int32) << 16)


def _unpack_bf16(w):
    lo = jax.lax.bitcast_convert_type(w << 16, jnp.float32)
    hi = jax.lax.bitcast_convert_type(w & jnp.int32(-65536), jnp.float32)
    return lo, hi


def _sc_scatter(table, src_ck, dst_ck, *, chunks_per_core, core_chunk_stride,
                src_3d, F=_D):
    cpt = chunks_per_core // NS
    rpt = _NPAD // NS
    G = 16
    n_super = cpt // G
    PW = F // 2
    params = pltpu.CompilerParams(
        needs_layout_passes=False, use_tc_tiling_on_sc=False)

    @functools.partial(
        pl.kernel,
        out_type=jax.ShapeDtypeStruct((NC, _NPAD, F), jnp.float32),
        mesh=_mesh(),
        compiler_params=params,
        scratch_types=[
            pltpu.VMEM_SHARED((_NPAD, F), jnp.float32),
            pltpu.VMEM((_CH, PW), jnp.int32),
            pltpu.VMEM((_CH, PW), jnp.int32),
            pltpu.VMEM((_CH, PW), jnp.int32),
            pltpu.VMEM((_CH, PW), jnp.int32),
            pltpu.VMEM((_CH, F), jnp.float32),
            pltpu.VMEM((_CH, F), jnp.float32),
            pltpu.VMEM((G, _CH), jnp.int32),
            pltpu.VMEM((G, _CH), jnp.int32),
            pltpu.SemaphoreType.DMA,
            pltpu.SemaphoreType.DMA,
            pltpu.SemaphoreType.DMA,
            pltpu.SemaphoreType.DMA,
            pltpu.SemaphoreType.DMA,
            pltpu.SemaphoreType.DMA,
        ],
    )
    def scat_kernel(table_hbm, src_hbm, dst_hbm, out_hbm, acc, ib0, ib1, ib2,
                    ib3, fb0, fb1, si, di, g0, g1, g2, g3, s0, s1):
        c = lax.axis_index("c")
        s = lax.axis_index("s")
        ibufs = [ib0, ib1, ib2, ib3]
        fbufs = [fb0, fb1]
        gsem = [g0, g1, g2, g3]
        ssem = [s0, s1]
        zeros16 = jnp.zeros((L,), jnp.float32)

        def zrow(k, carry):
            i = k // (F // L)
            j = k % (F // L)
            fb0[i, pl.ds(j * L, L)] = zeros16
            return carry

        lax.fori_loop(0, _CH * (F // L), zrow, 0)

        def zacc(i, carry):
            pltpu.sync_copy(fb0, acc.at[pl.ds(s * rpt + i * _CH, _CH)])
            return carry

        lax.fori_loop(0, rpt // _CH, zacc, 0)
        plsc.subcore_barrier()

        ck0 = c * core_chunk_stride + s * cpt
        ngrp = PW // L

        def gath(p):
            pltpu.async_copy(table_hbm.at[si.at[p]], ibufs[p % 4],
                             gsem[p % 4])

        def gath_wait(p):
            pltpu.make_async_copy(table_hbm.at[si.at[p]], ibufs[p % 4],
                                  gsem[p % 4]).wait()

        def scat_add(p):
            pltpu.async_copy(fbufs[p % 2], acc.at[di.at[p]], ssem[p % 2],
                             add=True)

        def scat_wait(p):
            pltpu.make_async_copy(fbufs[p % 2], acc.at[di.at[p]],
                                  ssem[p % 2]).wait()

        def convert(p):
            ib = ibufs[p % 4]
            fb = fbufs[p % 2]

            def cbody(g, carry):
                r = g // ngrp
                k = g % ngrp
                w = ib[r, pl.ds(k * L, L)]
                lo, hi = _unpack_bf16(w)
                fb[r, pl.ds(k * L, L)] = lo
                fb[r, pl.ds(PW + k * L, L)] = hi
                return carry

            lax.fori_loop(0, _CH * ngrp, cbody, 0)

        def super_body(u, carry):
            base = ck0 + u * G
            if src_3d:
                pltpu.sync_copy(src_hbm.at[c, pl.ds(base, G)], si)
            else:
                pltpu.sync_copy(src_hbm.at[pl.ds(base, G)], si)
            pltpu.sync_copy(dst_hbm.at[pl.ds(base, G)], di)
            for q in range(3):
                gath(q)
            for p in range(G):
                gath_wait(p)
                if p >= 2:
                    scat_wait(p - 2)
                convert(p)
                scat_add(p)
                if p + 3 < G:
                    gath(p + 3)
            scat_wait(G - 2)
            scat_wait(G - 1)
            return carry

        lax.fori_loop(0, n_super, super_body, 0)
        plsc.subcore_barrier()
        pltpu.sync_copy(acc.at[pl.ds(s * rpt, rpt)],
                        out_hbm.at[c, pl.ds(s * rpt, rpt)])

    return scat_kernel(table, src_ck, dst_ck)


def _tc_stage1(deg_parts, xpad, W1):
    def body(deg_ref, x_ref, w_ref, y_ref, dinv_ref):
        deg = jnp.sum(deg_ref[...], axis=0) + 1.0
        dinv = lax.rsqrt(deg)
        xw = jnp.dot(x_ref[...], w_ref[...],
                     preferred_element_type=jnp.float32)
        y = dinv[:, None] * xw
        y_ref[...] = _pack_bf16(y[:, :_D // 2], y[:, _D // 2:])[None]
        dinv_ref[...] = dinv[None, :]

    return pl.pallas_call(
        body,
        grid=(2, _NB),
        in_specs=[
            pl.BlockSpec((NW, _B), lambda h, i: (0, i)),
            pl.BlockSpec((_B, _D), lambda h, i: (i, 0)),
            pl.BlockSpec((_D, _D), lambda h, i: (0, h)),
        ],
        out_specs=[
            pl.BlockSpec((1, _B, _D // 2), lambda h, i: (h, i, 0)),
            pl.BlockSpec((1, _B), lambda h, i: (0, i)),
        ],
        out_shape=[
            jax.ShapeDtypeStruct((NC, _NPAD, _D // 2), jnp.int32),
            jax.ShapeDtypeStruct((1, _NPAD), jnp.float32),
        ],
    )(deg_parts, xpad, W1)


def _tc_stage2(agg1, y1, dinv, b1r, W2):
    F2 = _D // 2

    def body(a_ref, y_ref, dinv_ref, b_ref, w_ref, o_ref):
        dv = dinv_ref[0]
        b = b_ref[0]
        y0lo, y0hi = _unpack_bf16(y_ref[0])
        y1lo, y1hi = _unpack_bf16(y_ref[1])
        y0 = jnp.concatenate([y0lo, y0hi], axis=-1)
        y1 = jnp.concatenate([y1lo, y1hi], axis=-1)
        h_lo = jnp.maximum(
            dv[:, None] * (a_ref[0] + y0) + b[None, :_D], 0.0)
        h_hi = jnp.maximum(
            dv[:, None] * (a_ref[1] + y1) + b[None, _D:], 0.0)
        xw = (jnp.dot(h_lo, w_ref[0, :_D], preferred_element_type=jnp.float32)
              + jnp.dot(h_hi, w_ref[0, _D:],
                        preferred_element_type=jnp.float32))
        y2 = dv[:, None] * xw
        o_ref[...] = _pack_bf16(y2[:, :F2 // 2], y2[:, F2 // 2:])[None]

    return pl.pallas_call(
        body,
        grid=(2, _NB),
        in_specs=[
            pl.BlockSpec((NC, _B, _D), lambda h, i: (0, i, 0)),
            pl.BlockSpec((NC, _B, _D // 2), lambda h, i: (0, i, 0)),
            pl.BlockSpec((1, _B), lambda h, i: (0, i)),
            pl.BlockSpec((1, _H), lambda h, i: (0, 0)),
            pl.BlockSpec((1, _H, F2), lambda h, i: (h, 0, 0)),
        ],
        out_specs=pl.BlockSpec((1, _B, F2 // 2), lambda h, i: (h, i, 0)),
        out_shape=jax.ShapeDtypeStruct((NC, _NPAD, F2 // 2), jnp.int32),
    )(agg1, y1, dinv, b1r, W2)


def _tc_stage3(agg2, y2, dinv, b2r, wlr, blv):
    def body(a_ref, y_ref, dinv_ref, b_ref, wl_ref, bl_ref, s_ref):
        dv = dinv_ref[0]
        y0lo, y0hi = _unpack_bf16(y_ref[0])
        y1lo, y1hi = _unpack_bf16(y_ref[1])
        agg = jnp.concatenate(
            [a_ref[0] + jnp.concatenate([y0lo, y0hi], axis=-1),
             a_ref[1] + jnp.concatenate([y1lo, y1hi], axis=-1)], axis=-1)
        h2 = dv[:, None] * agg + b_ref[0][None, :]
        s = lax.dot_general(wl_ref[...], h2, (((1,), (1,)), ((), ())),
                            preferred_element_type=jnp.float32)
        s_ref[...] = s + bl_ref[...]

    return pl.pallas_call(
        body,
        grid=(_NB,),
        in_specs=[
            pl.BlockSpec((NC, _B, _D // 2), lambda i: (0, i, 0)),
            pl.BlockSpec((NC, _B, _D // 4), lambda i: (0, i, 0)),
            pl.BlockSpec((1, _B), lambda i: (0, i)),
            pl.BlockSpec((1, _D), lambda i: (0, 0)),
            pl.BlockSpec((2, _D), lambda i: (0, 0)),
            pl.BlockSpec((2, 1), lambda i: (0, 0)),
        ],
        out_specs=pl.BlockSpec((2, _B), lambda i: (0, i)),
        out_shape=jax.ShapeDtypeStruct((2, _NPAD), jnp.float32),
    )(agg2, y2, dinv, b2r, wlr, blv)


def _sc_mask(s_pair, m0, m1):
    @functools.partial(
        pl.kernel,
        out_type=jax.ShapeDtypeStruct((_M,), jnp.float32),
        mesh=_mesh(),
        compiler_params=_sc_params,
        scratch_types=[
            pltpu.VMEM((2, _NPAD), jnp.float32),
            pltpu.VMEM((_MW,), jnp.int32),
            pltpu.VMEM((_MW,), jnp.int32),
            pltpu.VMEM((_MW,), jnp.float32),
        ],
    )
    def mask_kernel(s_hbm, m0_hbm, m1_hbm, out_hbm, s_v, i0_v, i1_v, o_v):
        c = lax.axis_index("c")
        s = lax.axis_index("s")
        w = c * NS + s
        pltpu.sync_copy(s_hbm, s_v)
        pltpu.sync_copy(m0_hbm.at[pl.ds(w * _MW, _MW)], i0_v)
        pltpu.sync_copy(m1_hbm.at[pl.ds(w * _MW, _MW)], i1_v)
        z16 = jnp.zeros((L,), jnp.int32)
        o16 = jnp.ones((L,), jnp.int32)

        def body(j, carry):
            i0 = i0_v[pl.ds(j * L, L)]
            i1 = i1_v[pl.ds(j * L, L)]
            a = plsc.load_gather(s_v, [z16, i0])
            b = plsc.load_gather(s_v, [o16, i1])
            o_v[pl.ds(j * L, L)] = 1.0 / (1.0 + jnp.exp(-(a + b)))
            return carry

        lax.fori_loop(0, _MW // L, body, 0)
        pltpu.sync_copy(o_v, out_hbm.at[pl.ds(w * _MW, _MW)])

    return mask_kernel(s_pair, m0, m1)


def kernel(g, features, mask, W1, b1, W2, b2, Wl, bl):
    src = g[0].astype(jnp.int32)
    dst = g[1].astype(jnp.int32)
    padidx = jnp.full((_EPAD - _E,), _N, jnp.int32)
    src_p = jnp.concatenate([src, padidx])
    dst_p = jnp.concatenate([dst, padidx])
    src2 = jnp.stack([src_p, src_p + _NPAD])
    xpad = jnp.pad(features, ((0, _NPAD - _N), (0, 0)))

    n_ck = _EPAD // _CH
    _hbm = lambda a: pltpu.with_memory_space_constraint(a, pltpu.HBM)
    dst_ck = _hbm(dst_p.reshape(n_ck, _CH))
    src2_ck = _hbm(src2.reshape(NC, n_ck, _CH))
    src_ck = _hbm(src_p.reshape(n_ck, _CH))

    deg_parts = _sc_degree(dst_p)
    y1, dinv = _tc_stage1(deg_parts, xpad, W1)
    agg1 = _sc_scatter(y1.reshape(NC * _NPAD, _D // 2), src2_ck, dst_ck,
                       chunks_per_core=n_ck, core_chunk_stride=0,
                       src_3d=True)
    W2r = W2.reshape(_H, 2, _D // 2).transpose(1, 0, 2)
    y2 = _tc_stage2(agg1, y1, dinv, b1.reshape(1, _H), W2r)
    agg2 = _sc_scatter(y2.reshape(NC * _NPAD, _D // 4), src2_ck, dst_ck,
                       chunks_per_core=n_ck, core_chunk_stride=0,
                       src_3d=True, F=_D // 2)
    wlr = Wl[:, 0].reshape(2, _D)
    blv = jnp.pad(bl, (0, 1)).reshape(2, 1)
    s_pair = _tc_stage3(agg2, y2, dinv, b2.reshape(1, _D), wlr, blv)
    out = _sc_mask(s_pair, mask[:, 0].astype(jnp.int32),
                   mask[:, 1].astype(jnp.int32))
    return out.reshape(_M, 1)


# speedup vs baseline: 1.0648x; 1.0489x over previous
import functools

import jax
import jax.numpy as jnp
from jax import lax
from jax.experimental import pallas as pl
from jax.experimental.pallas import tpu as pltpu
from jax.experimental.pallas import tpu_sc as plsc

NC, NS, L = 2, 16, 16
NW = NC * NS

_N = 10000
_NPAD = 10240
_E = 320000
_EPAD = 327680
_CH = 80
_D = 128
_H = 256
_M = 65536
_MW = _M // NW
_B = 512
_NB = _NPAD // _B

def _mesh():
    return plsc.VectorSubcoreMesh(core_axis_name="c", subcore_axis_name="s",
                                  num_cores=NC, num_subcores=NS)
_sc_params = pltpu.CompilerParams(needs_layout_passes=False)


def _sc_degree(dst_pad):
    ew = _EPAD // NW

    @functools.partial(
        pl.kernel,
        out_type=jax.ShapeDtypeStruct((NW, _NPAD), jnp.float32),
        mesh=_mesh(),
        compiler_params=_sc_params,
        scratch_types=[
            pltpu.VMEM((_NPAD,), jnp.float32),
            pltpu.VMEM((ew,), jnp.int32),
        ],
    )
    def deg_kernel(dst_hbm, out_hbm, deg_v, idx_v):
        c = lax.axis_index("c")
        s = lax.axis_index("s")
        w = c * NS + s
        zeros16 = jnp.zeros((L,), jnp.float32)

        def zero_body(i, carry):
            deg_v[pl.ds(i * L, L)] = zeros16
            return carry

        lax.fori_loop(0, _NPAD // L, zero_body, 0)
        pltpu.sync_copy(dst_hbm.at[pl.ds(w * ew, ew)], idx_v)
        ones16 = jnp.ones((L,), jnp.float32)

        def add_body(j, carry):
            idx = idx_v[pl.ds(j * L, L)]
            plsc.addupdate_scatter(deg_v, [idx], ones16)
            return carry

        lax.fori_loop(0, ew // L, add_body, 0)
        pltpu.sync_copy(deg_v, out_hbm.at[w])

    return deg_kernel(dst_pad)


def _pack_bf16(lo, hi):
    lo16 = jax.lax.bitcast_convert_type(lo.astype(jnp.bfloat16), jnp.uint16)
    hi16 = jax.lax.bitcast_convert_type(hi.astype(jnp.bfloat16), jnp.uint16)
    return lo16.astype(jnp.int32) | (hi16.astype(jnp.int32) << 16)


def _unpack_bf16(w):
    lo = jax.lax.bitcast_convert_type(w << 16, jnp.float32)
    hi = jax.lax.bitcast_convert_type(w & jnp.int32(-65536), jnp.float32)
    return lo, hi


def _sc_scatter(table, src_ck, dst_ck, *, chunks_per_core, core_chunk_stride,
                src_3d, F=_D):
    cpt = chunks_per_core // NS
    rpt = _NPAD // NS
    G = 32
    n_super = cpt // G
    PW = F // 2
    params = pltpu.CompilerParams(
        needs_layout_passes=False, use_tc_tiling_on_sc=False)

    @functools.partial(
        pl.kernel,
        out_type=jax.ShapeDtypeStruct((NC, _NPAD, F), jnp.float32),
        mesh=_mesh(),
        compiler_params=params,
        scratch_types=[
            pltpu.VMEM_SHARED((_NPAD, F), jnp.float32),
            pltpu.VMEM((_CH, PW), jnp.int32),
            pltpu.VMEM((_CH, PW), jnp.int32),
            pltpu.VMEM((_CH, PW), jnp.int32),
            pltpu.VMEM((_CH, PW), jnp.int32),
            pltpu.VMEM((_CH, F), jnp.float32),
            pltpu.VMEM((_CH, F), jnp.float32),
            pltpu.VMEM((G, _CH), jnp.int32),
            pltpu.VMEM((G, _CH), jnp.int32),
            pltpu.SemaphoreType.DMA,
            pltpu.SemaphoreType.DMA,
            pltpu.SemaphoreType.DMA,
            pltpu.SemaphoreType.DMA,
            pltpu.SemaphoreType.DMA,
            pltpu.SemaphoreType.DMA,
        ],
    )
    def scat_kernel(table_hbm, src_hbm, dst_hbm, out_hbm, acc, ib0, ib1, ib2,
                    ib3, fb0, fb1, si, di, g0, g1, g2, g3, s0, s1):
        c = lax.axis_index("c")
        s = lax.axis_index("s")
        ibufs = [ib0, ib1, ib2, ib3]
        fbufs = [fb0, fb1]
        gsem = [g0, g1, g2, g3]
        ssem = [s0, s1]
        zeros16 = jnp.zeros((L,), jnp.float32)

        def zrow(k, carry):
            i = k // (F // L)
            j = k % (F // L)
            fb0[i, pl.ds(j * L, L)] = zeros16
            return carry

        lax.fori_loop(0, _CH * (F // L), zrow, 0)

        def zacc(i, carry):
            pltpu.sync_copy(fb0, acc.at[pl.ds(s * rpt + i * _CH, _CH)])
            return carry

        lax.fori_loop(0, rpt // _CH, zacc, 0)
        plsc.subcore_barrier()

        ck0 = c * core_chunk_stride + s * cpt
        ngrp = PW // L

        def gath(p):
            pltpu.async_copy(table_hbm.at[si.at[p]], ibufs[p % 4],
                             gsem[p % 4])

        def gath_wait(p):
            pltpu.make_async_copy(table_hbm.at[si.at[p]], ibufs[p % 4],
                                  gsem[p % 4]).wait()

        def scat_add(p):
            pltpu.async_copy(fbufs[p % 2], acc.at[di.at[p]], ssem[p % 2],
                             add=True)

        def scat_wait(p):
            pltpu.make_async_copy(fbufs[p % 2], acc.at[di.at[p]],
                                  ssem[p % 2]).wait()

        def convert(p):
            ib = ibufs[p % 4]
            fb = fbufs[p % 2]

            def cbody(g, carry):
                r = g // ngrp
                k = g % ngrp
                w = ib[r, pl.ds(k * L, L)]
                lo, hi = _unpack_bf16(w)
                fb[r, pl.ds(k * L, L)] = lo
                fb[r, pl.ds(PW + k * L, L)] = hi
                return carry

            lax.fori_loop(0, _CH * ngrp, cbody, 0)

        def super_body(u, carry):
            base = ck0 + u * G
            if src_3d:
                pltpu.sync_copy(src_hbm.at[c, pl.ds(base, G)], si)
            else:
                pltpu.sync_copy(src_hbm.at[pl.ds(base, G)], si)
            pltpu.sync_copy(dst_hbm.at[pl.ds(base, G)], di)
            for q in range(3):
                gath(q)
            for p in range(G):
                gath_wait(p)
                if p >= 2:
                    scat_wait(p - 2)
                convert(p)
                scat_add(p)
                if p + 3 < G:
                    gath(p + 3)
            scat_wait(G - 2)
            scat_wait(G - 1)
            return carry

        lax.fori_loop(0, n_super, super_body, 0)
        plsc.subcore_barrier()
        pltpu.sync_copy(acc.at[pl.ds(s * rpt, rpt)],
                        out_hbm.at[c, pl.ds(s * rpt, rpt)])

    return scat_kernel(table, src_ck, dst_ck)


def _tc_stage1(deg_parts, xpad, W1):
    def body(deg_ref, x_ref, w_ref, y_ref, dinv_ref):
        deg = jnp.sum(deg_ref[...], axis=0) + 1.0
        dinv = lax.rsqrt(deg)
        xw = jnp.dot(x_ref[...], w_ref[...],
                     preferred_element_type=jnp.float32)
        y = dinv[:, None] * xw
        y_ref[...] = _pack_bf16(y[:, :_D // 2], y[:, _D // 2:])[None]
        dinv_ref[...] = dinv[None, :]

    return pl.pallas_call(
        body,
        grid=(2, _NB),
        in_specs=[
            pl.BlockSpec((NW, _B), lambda h, i: (0, i)),
            pl.BlockSpec((_B, _D), lambda h, i: (i, 0)),
            pl.BlockSpec((_D, _D), lambda h, i: (0, h)),
        ],
        out_specs=[
            pl.BlockSpec((1, _B, _D // 2), lambda h, i: (h, i, 0)),
            pl.BlockSpec((1, _B), lambda h, i: (0, i)),
        ],
        out_shape=[
            jax.ShapeDtypeStruct((NC, _NPAD, _D // 2), jnp.int32),
            jax.ShapeDtypeStruct((1, _NPAD), jnp.float32),
        ],
    )(deg_parts, xpad, W1)


def _tc_stage2(agg1, y1, dinv, b1r, W2):
    F2 = _D // 2

    def body(a_ref, y_ref, dinv_ref, b_ref, w_ref, o_ref):
        dv = dinv_ref[0]
        b = b_ref[0]
        y0lo, y0hi = _unpack_bf16(y_ref[0])
        y1lo, y1hi = _unpack_bf16(y_ref[1])
        y0 = jnp.concatenate([y0lo, y0hi], axis=-1)
        y1 = jnp.concatenate([y1lo, y1hi], axis=-1)
        h_lo = jnp.maximum(
            dv[:, None] * (a_ref[0] + y0) + b[None, :_D], 0.0)
        h_hi = jnp.maximum(
            dv[:, None] * (a_ref[1] + y1) + b[None, _D:], 0.0)
        xw = (jnp.dot(h_lo, w_ref[0, :_D], preferred_element_type=jnp.float32)
              + jnp.dot(h_hi, w_ref[0, _D:],
                        preferred_element_type=jnp.float32))
        y2 = dv[:, None] * xw
        o_ref[...] = _pack_bf16(y2[:, :F2 // 2], y2[:, F2 // 2:])[None]

    return pl.pallas_call(
        body,
        grid=(2, _NB),
        in_specs=[
            pl.BlockSpec((NC, _B, _D), lambda h, i: (0, i, 0)),
            pl.BlockSpec((NC, _B, _D // 2), lambda h, i: (0, i, 0)),
            pl.BlockSpec((1, _B), lambda h, i: (0, i)),
            pl.BlockSpec((1, _H), lambda h, i: (0, 0)),
            pl.BlockSpec((1, _H, F2), lambda h, i: (h, 0, 0)),
        ],
        out_specs=pl.BlockSpec((1, _B, F2 // 2), lambda h, i: (h, i, 0)),
        out_shape=jax.ShapeDtypeStruct((NC, _NPAD, F2 // 2), jnp.int32),
    )(agg1, y1, dinv, b1r, W2)


def _tc_stage3(agg2, y2, dinv, b2r, wlr, blv):
    def body(a_ref, y_ref, dinv_ref, b_ref, wl_ref, bl_ref, s_ref):
        dv = dinv_ref[0]
        y0lo, y0hi = _unpack_bf16(y_ref[0])
        y1lo, y1hi = _unpack_bf16(y_ref[1])
        agg = jnp.concatenate(
            [a_ref[0] + jnp.concatenate([y0lo, y0hi], axis=-1),
             a_ref[1] + jnp.concatenate([y1lo, y1hi], axis=-1)], axis=-1)
        h2 = dv[:, None] * agg + b_ref[0][None, :]
        s = lax.dot_general(wl_ref[...], h2, (((1,), (1,)), ((), ())),
                            preferred_element_type=jnp.float32)
        s_ref[...] = s + bl_ref[...]

    return pl.pallas_call(
        body,
        grid=(_NB,),
        in_specs=[
            pl.BlockSpec((NC, _B, _D // 2), lambda i: (0, i, 0)),
            pl.BlockSpec((NC, _B, _D // 4), lambda i: (0, i, 0)),
            pl.BlockSpec((1, _B), lambda i: (0, i)),
            pl.BlockSpec((1, _D), lambda i: (0, 0)),
            pl.BlockSpec((2, _D), lambda i: (0, 0)),
            pl.BlockSpec((2, 1), lambda i: (0, 0)),
        ],
        out_specs=pl.BlockSpec((2, _B), lambda i: (0, i)),
        out_shape=jax.ShapeDtypeStruct((2, _NPAD), jnp.float32),
    )(agg2, y2, dinv, b2r, wlr, blv)


def _sc_mask(s_pair, m0, m1):
    @functools.partial(
        pl.kernel,
        out_type=jax.ShapeDtypeStruct((_M,), jnp.float32),
        mesh=_mesh(),
        compiler_params=_sc_params,
        scratch_types=[
            pltpu.VMEM((2, _NPAD), jnp.float32),
            pltpu.VMEM((_MW,), jnp.int32),
            pltpu.VMEM((_MW,), jnp.int32),
            pltpu.VMEM((_MW,), jnp.float32),
        ],
    )
    def mask_kernel(s_hbm, m0_hbm, m1_hbm, out_hbm, s_v, i0_v, i1_v, o_v):
        c = lax.axis_index("c")
        s = lax.axis_index("s")
        w = c * NS + s
        pltpu.sync_copy(s_hbm, s_v)
        pltpu.sync_copy(m0_hbm.at[pl.ds(w * _MW, _MW)], i0_v)
        pltpu.sync_copy(m1_hbm.at[pl.ds(w * _MW, _MW)], i1_v)
        z16 = jnp.zeros((L,), jnp.int32)
        o16 = jnp.ones((L,), jnp.int32)

        def body(j, carry):
            i0 = i0_v[pl.ds(j * L, L)]
            i1 = i1_v[pl.ds(j * L, L)]
            a = plsc.load_gather(s_v, [z16, i0])
            b = plsc.load_gather(s_v, [o16, i1])
            o_v[pl.ds(j * L, L)] = 1.0 / (1.0 + jnp.exp(-(a + b)))
            return carry

        lax.fori_loop(0, _MW // L, body, 0)
        pltpu.sync_copy(o_v, out_hbm.at[pl.ds(w * _MW, _MW)])

    return mask_kernel(s_pair, m0, m1)


def kernel(g, features, mask, W1, b1, W2, b2, Wl, bl):
    src = g[0].astype(jnp.int32)
    dst = g[1].astype(jnp.int32)
    padidx = jnp.full((_EPAD - _E,), _N, jnp.int32)
    src_p = jnp.concatenate([src, padidx])
    dst_p = jnp.concatenate([dst, padidx])
    src2 = jnp.stack([src_p, src_p + _NPAD])
    xpad = jnp.pad(features, ((0, _NPAD - _N), (0, 0)))

    n_ck = _EPAD // _CH
    _hbm = lambda a: pltpu.with_memory_space_constraint(a, pltpu.HBM)
    dst_ck = _hbm(dst_p.reshape(n_ck, _CH))
    src2_ck = _hbm(src2.reshape(NC, n_ck, _CH))
    src_ck = _hbm(src_p.reshape(n_ck, _CH))

    deg_parts = _sc_degree(dst_p)
    y1, dinv = _tc_stage1(deg_parts, xpad, W1)
    agg1 = _sc_scatter(y1.reshape(NC * _NPAD, _D // 2), src2_ck, dst_ck,
                       chunks_per_core=n_ck, core_chunk_stride=0,
                       src_3d=True)
    W2r = W2.reshape(_H, 2, _D // 2).transpose(1, 0, 2)
    y2 = _tc_stage2(agg1, y1, dinv, b1.reshape(1, _H), W2r)
    agg2 = _sc_scatter(y2.reshape(NC * _NPAD, _D // 4), src2_ck, dst_ck,
                       chunks_per_core=n_ck, core_chunk_stride=0,
                       src_3d=True, F=_D // 2)
    wlr = Wl[:, 0].reshape(2, _D)
    blv = jnp.pad(bl, (0, 1)).reshape(2, 1)
    s_pair = _tc_stage3(agg2, y2, dinv, b2.reshape(1, _D), wlr, blv)
    out = _sc_mask(s_pair, mask[:, 0].astype(jnp.int32),
                   mask[:, 1].astype(jnp.int32))
    return out.reshape(_M, 1)
